# baseline (speedup 1.0000x reference)
import functools
import jax
import jax.numpy as jnp
from jax import lax
from jax.experimental import pallas as pl
from jax.experimental.pallas import tpu as pltpu
from jax.experimental.pallas import tpu_sc as plsc

N = 10000
E = 320000
P = 32768
D = 128
NRELS = 1000
NC, NS = 2, 16
NW = NC * NS
K1 = 80
K3 = 64
NP = 10240
RPT = NP // NS


def _sc_mesh():
    return plsc.VectorSubcoreMesh(
        core_axis_name="c", subcore_axis_name="s", num_cores=NC, num_subcores=NS
    )


def _gs_body(tbl_hbm, g_hbm, t_hbm, z2d_hbm, out_hbm,
             idx_g, idx_s, rows, acc):
    c = lax.axis_index("c")
    s = lax.axis_index("s")
    r0 = s * RPT
    pltpu.sync_copy(z2d_hbm.at[pl.ds(r0, RPT)], acc.at[pl.ds(r0, RPT)])
    plsc.subcore_barrier()

    iters = E // NW // K1
    wid = c * NS + s
    pltpu.sync_copy(g_hbm.at[wid], idx_g)
    pltpu.sync_copy(t_hbm.at[wid], idx_s)

    @pl.loop(0, iters)
    def _(i):
        pltpu.sync_copy(tbl_hbm.at[idx_g.at[i]], rows)
        pltpu.sync_copy(rows, acc.at[idx_s.at[i]], add=True)

    plsc.subcore_barrier()
    pltpu.sync_copy(acc.at[pl.ds(r0, RPT)], out_hbm.at[c, pl.ds(r0, RPT)])


def _gather_scatter(table, gidx3, sidx3, z2d):
    f32 = jnp.float32
    return pl.kernel(
        _gs_body,
        out_type=jax.ShapeDtypeStruct((NC, NP, D), f32),
        mesh=_sc_mesh(),
        scratch_types=(
            pltpu.VMEM((E // NW // K1, K1), jnp.int32),
            pltpu.VMEM((E // NW // K1, K1), jnp.int32),
            pltpu.VMEM((K1, D), f32),
            pltpu.VMEM_SHARED((NP, D), f32),
        ),
    )(table, gidx3, sidx3, z2d)


def _score_body(x_hbm, rel_hbm, ph_hbm, pt_hbm, pr_hbm, pn_hbm,
                sp_hbm, sn_hbm,
                ih, it, ir, inn, hrows, trows, rrows, nrows, stg_p, stg_n):
    c = lax.axis_index("c")
    s = lax.axis_index("s")
    iters = P // NW // K3
    wid = c * NS + s
    row_base = wid * iters
    pltpu.sync_copy(ph_hbm.at[wid], ih)
    pltpu.sync_copy(pt_hbm.at[wid], it)
    pltpu.sync_copy(pr_hbm.at[wid], ir)
    pltpu.sync_copy(pn_hbm.at[wid], inn)

    @pl.loop(0, iters)
    def _(i):
        pltpu.sync_copy(x_hbm.at[ih.at[i]], hrows)
        pltpu.sync_copy(x_hbm.at[it.at[i]], trows)
        pltpu.sync_copy(rel_hbm.at[ir.at[i]], rrows)
        pltpu.sync_copy(x_hbm.at[inn.at[i]], nrows)

        @pl.loop(0, K3)
        def _(j):
            accp = jnp.zeros((16,), jnp.float32)
            accn = jnp.zeros((16,), jnp.float32)
            for d in range(D // 16):
                sl = pl.ds(16 * d, 16)
                u = hrows[j, sl] + rrows[j, sl]
                dp = u - trows[j, sl]
                dn = u - nrows[j, sl]
                accp = accp + dp * dp
                accn = accn + dn * dn
            stg_p[j, :] = accp
            stg_n[j, :] = accn

        pltpu.sync_copy(stg_p, sp_hbm.at[pl.ds((row_base + i) * K3, K3)])
        pltpu.sync_copy(stg_n, sn_hbm.at[pl.ds((row_base + i) * K3, K3)])


def _score(x2, rel_emb, ph3, pt3, pr3, pn3):
    f32 = jnp.float32
    return pl.kernel(
        _score_body,
        out_type=(
            jax.ShapeDtypeStruct((P, 16), f32),
            jax.ShapeDtypeStruct((P, 16), f32),
        ),
        mesh=_sc_mesh(),
        scratch_types=(
            pltpu.VMEM((P // NW // K3, K3), jnp.int32),
            pltpu.VMEM((P // NW // K3, K3), jnp.int32),
            pltpu.VMEM((P // NW // K3, K3), jnp.int32),
            pltpu.VMEM((P // NW // K3, K3), jnp.int32),
            pltpu.VMEM((K3, D), f32),
            pltpu.VMEM((K3, D), f32),
            pltpu.VMEM((K3, D), f32),
            pltpu.VMEM((K3, D), f32),
            pltpu.VMEM((K3, 16), f32),
            pltpu.VMEM((K3, 16), f32),
        ),
    )(x2, rel_emb, ph3, pt3, pr3, pn3)


RB = 1000


def _fuse1_kernel(feat, a0, a1, b0, b1, cs0, cs1, cd0, cd1,
                  w_out, bo, w_in, bi, w1, y1):
    cs = cs0[:, 0:1] + cs1[:, 0:1]
    cd = cd0[:, 0:1] + cd1[:, 0:1]
    w1a = w1[0:D, :]
    w1b = w1[D:2 * D, :]
    w1c = w1[2 * D:3 * D, :]
    dot = functools.partial(jnp.dot, preferred_element_type=jnp.float32)
    acc = dot(feat[...], w1a)
    acc += dot(a0[...] + a1[...], dot(w_out[...], w1b))
    acc += dot(b0[...] + b1[...], dot(w_in[...], w1c))
    acc += cs * dot(bo[...], w1b)
    acc += cd * dot(bi[...], w1c)
    out_norm = lax.rsqrt(jnp.maximum(cs, 1.0))
    y1[...] = acc * out_norm


def _fuse1(feat, a0, a1, b0, b1, cs0, cs1, cd0, cd1, w_out, bo, w_in, bi, w1):
    f32 = jnp.float32
    grid = N // RB
    row = lambda i: (i, 0)
    fixed = lambda i: (0, 0)
    return pl.pallas_call(
        _fuse1_kernel,
        grid=(grid,),
        in_specs=[
            pl.BlockSpec((RB, D), row),
            pl.BlockSpec((RB, D), row),
            pl.BlockSpec((RB, D), row),
            pl.BlockSpec((RB, D), row),
            pl.BlockSpec((RB, D), row),
            pl.BlockSpec((RB, D), row),
            pl.BlockSpec((RB, D), row),
            pl.BlockSpec((RB, D), row),
            pl.BlockSpec((RB, D), row),
            pl.BlockSpec((D, D), fixed),
            pl.BlockSpec((1, D), fixed),
            pl.BlockSpec((D, D), fixed),
            pl.BlockSpec((1, D), fixed),
            pl.BlockSpec((3 * D, D), fixed),
        ],
        out_specs=pl.BlockSpec((RB, D), row),
        out_shape=jax.ShapeDtypeStruct((N, D), f32),
    )(feat, a0, a1, b0, b1, cs0, cs1, cd0, cd1, w_out, bo, w_in, bi, w1)


def _fuse2_kernel(m0, m1, cs0, cs1, cd0, cd1, b1, w2, y2):
    in_norm = lax.rsqrt(jnp.maximum(cd0[:, 0:1] + cd1[:, 0:1], 1.0))
    x1 = jax.nn.relu((m0[...] + m1[...]) * in_norm + b1[...])
    out_norm = lax.rsqrt(jnp.maximum(cs0[:, 0:1] + cs1[:, 0:1], 1.0))
    y2[...] = jnp.dot(x1, w2[...], preferred_element_type=jnp.float32) * out_norm


def _fuse2(m0, m1, cs0, cs1, cd0, cd1, b1, w2):
    f32 = jnp.float32
    grid = N // RB
    row = lambda i: (i, 0)
    fixed = lambda i: (0, 0)
    return pl.pallas_call(
        _fuse2_kernel,
        grid=(grid,),
        in_specs=[
            pl.BlockSpec((RB, D), row),
            pl.BlockSpec((RB, D), row),
            pl.BlockSpec((RB, D), row),
            pl.BlockSpec((RB, D), row),
            pl.BlockSpec((RB, D), row),
            pl.BlockSpec((RB, D), row),
            pl.BlockSpec((1, D), fixed),
            pl.BlockSpec((D, D), fixed),
        ],
        out_specs=pl.BlockSpec((RB, D), row),
        out_shape=jax.ShapeDtypeStruct((N, D), f32),
    )(m0, m1, cs0, cs1, cd0, cd1, b1, w2)


def _fuse3_kernel(m0, m1, cd0, cd1, b2, x2):
    in_norm = lax.rsqrt(jnp.maximum(cd0[:, 0:1] + cd1[:, 0:1], 1.0))
    x2[...] = jax.nn.relu((m0[...] + m1[...]) * in_norm + b2[...])


def _fuse3(m0, m1, cd0, cd1, b2):
    f32 = jnp.float32
    grid = N // RB
    row = lambda i: (i, 0)
    fixed = lambda i: (0, 0)
    return pl.pallas_call(
        _fuse3_kernel,
        grid=(grid,),
        in_specs=[
            pl.BlockSpec((RB, D), row),
            pl.BlockSpec((RB, D), row),
            pl.BlockSpec((RB, D), row),
            pl.BlockSpec((RB, D), row),
            pl.BlockSpec((1, D), fixed),
        ],
        out_specs=pl.BlockSpec((RB, D), row),
        out_shape=jax.ShapeDtypeStruct((N, D), f32),
    )(m0, m1, cd0, cd1, b2)


PB = 2048


def _sqrt_kernel(sp, sn, op, on):
    op[...] = jnp.sqrt(jnp.sum(sp[...], axis=1, keepdims=True))
    on[...] = jnp.sqrt(jnp.sum(sn[...], axis=1, keepdims=True))


def _sqrt2(sp16, sn16):
    f32 = jnp.float32
    grid = P // PB
    row = lambda i: (i, 0)
    return pl.pallas_call(
        _sqrt_kernel,
        grid=(grid,),
        in_specs=[pl.BlockSpec((PB, 16), row), pl.BlockSpec((PB, 16), row)],
        out_specs=(pl.BlockSpec((PB, 1), row), pl.BlockSpec((PB, 1), row)),
        out_shape=(jax.ShapeDtypeStruct((P, 1), f32),
                   jax.ShapeDtypeStruct((P, 1), f32)),
    )(sp16, sn16)


def kernel(input_feat, edge_index, edge_type, pos_edge_index, pos_edge_type,
           neg_dst, rel_emb, W_out, b_out, W_in, b_in, W1, b1, W2, b2):
    src_m = edge_index[0].reshape(NW, E // NW // K1, K1)
    dst_m = edge_index[1].reshape(NW, E // NW // K1, K1)
    et_m = edge_type.reshape(NW, E // NW // K1, K1)
    z2d = jnp.zeros((NP, D), jnp.float32)
    zidx = jnp.zeros((NW, E // NW // K1, K1), jnp.int32)
    ones_tbl = jnp.ones((N, D), jnp.float32)
    rel_pad = jnp.concatenate(
        [rel_emb, jnp.zeros((N - NRELS, D), jnp.float32)], axis=0)

    cs = _gather_scatter(ones_tbl, zidx, src_m, z2d)
    cd = _gather_scatter(ones_tbl, zidx, dst_m, z2d)
    ra = _gather_scatter(rel_pad, et_m, src_m, z2d)
    rb = _gather_scatter(rel_pad, et_m, dst_m, z2d)
    y1 = _fuse1(input_feat, ra[0], ra[1], rb[0], rb[1],
                cs[0], cs[1], cd[0], cd[1],
                W_out, b_out.reshape(1, D), W_in, b_in.reshape(1, D), W1)
    m = _gather_scatter(y1, src_m, dst_m, z2d)
    y2 = _fuse2(m[0], m[1], cs[0], cs[1], cd[0], cd[1], b1.reshape(1, D), W2)
    n = _gather_scatter(y2, src_m, dst_m, z2d)
    x2 = _fuse3(n[0], n[1], cd[0], cd[1], b2.reshape(1, D))

    ph3 = pos_edge_index[0].reshape(NW, P // NW // K3, K3)
    pt3 = pos_edge_index[1].reshape(NW, P // NW // K3, K3)
    pr3 = pos_edge_type.reshape(NW, P // NW // K3, K3)
    pn3 = neg_dst.reshape(NW, P // NW // K3, K3)
    sp16, sn16 = _score(x2, rel_emb, ph3, pt3, pr3, pn3)
    pos, neg = _sqrt2(sp16, sn16)
    return pos.reshape(P), neg.reshape(P)


# speedup vs baseline: 16.7174x; 16.7174x over previous
import functools
import jax
import jax.numpy as jnp
from jax import lax
from jax.experimental import pallas as pl
from jax.experimental.pallas import tpu as pltpu
from jax.experimental.pallas import tpu_sc as plsc

N = 10000
E = 320000
P = 32768
D = 128
NRELS = 1000
NC, NS = 2, 16
NW = NC * NS
K1 = 80
K3 = 64
NP = 10240
RPT = NP // NS


def _sc_mesh():
    return plsc.VectorSubcoreMesh(
        core_axis_name="c", subcore_axis_name="s", num_cores=NC, num_subcores=NS
    )


def _gs_body(tbl_hbm, g_hbm, t_hbm, z2d_hbm, out_hbm,
             idx_g, idx_s, rows, acc):
    c = lax.axis_index("c")
    s = lax.axis_index("s")
    r0 = s * RPT
    pltpu.sync_copy(z2d_hbm.at[pl.ds(r0, RPT)], acc.at[pl.ds(r0, RPT)])
    plsc.subcore_barrier()

    iters = E // NW // K1
    wid = c * NS + s
    pltpu.sync_copy(g_hbm.at[wid], idx_g)
    pltpu.sync_copy(t_hbm.at[wid], idx_s)

    @pl.loop(0, iters)
    def _(i):
        pltpu.sync_copy(tbl_hbm.at[idx_g.at[i]], rows)
        pltpu.sync_copy(rows, acc.at[idx_s.at[i]], add=True)

    plsc.subcore_barrier()
    pltpu.sync_copy(acc.at[pl.ds(r0, RPT)], out_hbm.at[c, pl.ds(r0, RPT)])


def _gather_scatter(table, gidx3, sidx3, z2d):
    f32 = jnp.float32
    return pl.kernel(
        _gs_body,
        out_type=jax.ShapeDtypeStruct((NC, NP, D), f32),
        mesh=_sc_mesh(),
        scratch_types=(
            pltpu.VMEM((E // NW // K1, K1), jnp.int32),
            pltpu.VMEM((E // NW // K1, K1), jnp.int32),
            pltpu.VMEM((K1, D), f32),
            pltpu.VMEM_SHARED((NP, D), f32),
        ),
    )(table, gidx3, sidx3, z2d)


def _score_body(x_hbm, rel_hbm, ph_hbm, pt_hbm, pr_hbm, pn_hbm,
                sp_hbm, sn_hbm,
                ih, it, ir, inn, hrows, trows, rrows, nrows, stg_p, stg_n):
    c = lax.axis_index("c")
    s = lax.axis_index("s")
    iters = P // NW // K3
    wid = c * NS + s
    row_base = wid * iters
    pltpu.sync_copy(ph_hbm.at[wid], ih)
    pltpu.sync_copy(pt_hbm.at[wid], it)
    pltpu.sync_copy(pr_hbm.at[wid], ir)
    pltpu.sync_copy(pn_hbm.at[wid], inn)

    @pl.loop(0, iters)
    def _(i):
        pltpu.sync_copy(x_hbm.at[ih.at[i]], hrows)
        pltpu.sync_copy(x_hbm.at[it.at[i]], trows)
        pltpu.sync_copy(rel_hbm.at[ir.at[i]], rrows)
        pltpu.sync_copy(x_hbm.at[inn.at[i]], nrows)

        @pl.loop(0, K3)
        def _(j):
            accp = jnp.zeros((16,), jnp.float32)
            accn = jnp.zeros((16,), jnp.float32)
            for d in range(D // 16):
                sl = pl.ds(16 * d, 16)
                u = hrows[j, sl] + rrows[j, sl]
                dp = u - trows[j, sl]
                dn = u - nrows[j, sl]
                accp = accp + dp * dp
                accn = accn + dn * dn
            stg_p[j, :] = accp
            stg_n[j, :] = accn

        pltpu.sync_copy(stg_p, sp_hbm.at[pl.ds((row_base + i) * K3, K3)])
        pltpu.sync_copy(stg_n, sn_hbm.at[pl.ds((row_base + i) * K3, K3)])


def _score(x2, rel_emb, ph3, pt3, pr3, pn3):
    f32 = jnp.float32
    return pl.kernel(
        _score_body,
        out_type=(
            jax.ShapeDtypeStruct((P, 16), f32),
            jax.ShapeDtypeStruct((P, 16), f32),
        ),
        mesh=_sc_mesh(),
        scratch_types=(
            pltpu.VMEM((P // NW // K3, K3), jnp.int32),
            pltpu.VMEM((P // NW // K3, K3), jnp.int32),
            pltpu.VMEM((P // NW // K3, K3), jnp.int32),
            pltpu.VMEM((P // NW // K3, K3), jnp.int32),
            pltpu.VMEM((K3, D), f32),
            pltpu.VMEM((K3, D), f32),
            pltpu.VMEM((K3, D), f32),
            pltpu.VMEM((K3, D), f32),
            pltpu.VMEM((K3, 16), f32),
            pltpu.VMEM((K3, 16), f32),
        ),
    )(x2, rel_emb, ph3, pt3, pr3, pn3)


RB = 1000


def _fuse1_kernel(feat, a0, a1, b0, b1, cs0, cs1, cd0, cd1,
                  w_out, bo, w_in, bi, w1, y1):
    cs = cs0[:, 0:1] + cs1[:, 0:1]
    cd = cd0[:, 0:1] + cd1[:, 0:1]
    w1a = w1[0:D, :]
    w1b = w1[D:2 * D, :]
    w1c = w1[2 * D:3 * D, :]
    dot = functools.partial(jnp.dot, preferred_element_type=jnp.float32)
    acc = dot(feat[...], w1a)
    acc += dot(a0[...] + a1[...], dot(w_out[...], w1b))
    acc += dot(b0[...] + b1[...], dot(w_in[...], w1c))
    acc += cs * dot(bo[...], w1b)
    acc += cd * dot(bi[...], w1c)
    out_norm = lax.rsqrt(jnp.maximum(cs, 1.0))
    y1[...] = acc * out_norm


def _fuse1(feat, a0, a1, b0, b1, cs0, cs1, cd0, cd1, w_out, bo, w_in, bi, w1):
    f32 = jnp.float32
    grid = N // RB
    row = lambda i: (i, 0)
    fixed = lambda i: (0, 0)
    return pl.pallas_call(
        _fuse1_kernel,
        grid=(grid,),
        in_specs=[
            pl.BlockSpec((RB, D), row),
            pl.BlockSpec((RB, D), row),
            pl.BlockSpec((RB, D), row),
            pl.BlockSpec((RB, D), row),
            pl.BlockSpec((RB, D), row),
            pl.BlockSpec((RB, D), row),
            pl.BlockSpec((RB, D), row),
            pl.BlockSpec((RB, D), row),
            pl.BlockSpec((RB, D), row),
            pl.BlockSpec((D, D), fixed),
            pl.BlockSpec((1, D), fixed),
            pl.BlockSpec((D, D), fixed),
            pl.BlockSpec((1, D), fixed),
            pl.BlockSpec((3 * D, D), fixed),
        ],
        out_specs=pl.BlockSpec((RB, D), row),
        out_shape=jax.ShapeDtypeStruct((N, D), f32),
    )(feat, a0, a1, b0, b1, cs0, cs1, cd0, cd1, w_out, bo, w_in, bi, w1)


def _fuse2_kernel(m0, m1, cs0, cs1, cd0, cd1, b1, w2, y2):
    in_norm = lax.rsqrt(jnp.maximum(cd0[:, 0:1] + cd1[:, 0:1], 1.0))
    x1 = jax.nn.relu((m0[...] + m1[...]) * in_norm + b1[...])
    out_norm = lax.rsqrt(jnp.maximum(cs0[:, 0:1] + cs1[:, 0:1], 1.0))
    y2[...] = jnp.dot(x1, w2[...], preferred_element_type=jnp.float32) * out_norm


def _fuse2(m0, m1, cs0, cs1, cd0, cd1, b1, w2):
    f32 = jnp.float32
    grid = N // RB
    row = lambda i: (i, 0)
    fixed = lambda i: (0, 0)
    return pl.pallas_call(
        _fuse2_kernel,
        grid=(grid,),
        in_specs=[
            pl.BlockSpec((RB, D), row),
            pl.BlockSpec((RB, D), row),
            pl.BlockSpec((RB, D), row),
            pl.BlockSpec((RB, D), row),
            pl.BlockSpec((RB, D), row),
            pl.BlockSpec((RB, D), row),
            pl.BlockSpec((1, D), fixed),
            pl.BlockSpec((D, D), fixed),
        ],
        out_specs=pl.BlockSpec((RB, D), row),
        out_shape=jax.ShapeDtypeStruct((N, D), f32),
    )(m0, m1, cs0, cs1, cd0, cd1, b1, w2)


def _fuse3_kernel(m0, m1, cd0, cd1, b2, x2):
    in_norm = lax.rsqrt(jnp.maximum(cd0[:, 0:1] + cd1[:, 0:1], 1.0))
    x2[...] = jax.nn.relu((m0[...] + m1[...]) * in_norm + b2[...])


def _fuse3(m0, m1, cd0, cd1, b2):
    f32 = jnp.float32
    grid = N // RB
    row = lambda i: (i, 0)
    fixed = lambda i: (0, 0)
    return pl.pallas_call(
        _fuse3_kernel,
        grid=(grid,),
        in_specs=[
            pl.BlockSpec((RB, D), row),
            pl.BlockSpec((RB, D), row),
            pl.BlockSpec((RB, D), row),
            pl.BlockSpec((RB, D), row),
            pl.BlockSpec((1, D), fixed),
        ],
        out_specs=pl.BlockSpec((RB, D), row),
        out_shape=jax.ShapeDtypeStruct((N, D), f32),
    )(m0, m1, cd0, cd1, b2)


PB = 2048


def _sqrt_kernel(sp, sn, op, on):
    op[...] = jnp.sqrt(jnp.sum(sp[...], axis=1, keepdims=True))
    on[...] = jnp.sqrt(jnp.sum(sn[...], axis=1, keepdims=True))


def _sqrt2(sp16, sn16):
    f32 = jnp.float32
    grid = P // PB
    row = lambda i: (i, 0)
    return pl.pallas_call(
        _sqrt_kernel,
        grid=(grid,),
        in_specs=[pl.BlockSpec((PB, 16), row), pl.BlockSpec((PB, 16), row)],
        out_specs=(pl.BlockSpec((PB, 1), row), pl.BlockSpec((PB, 1), row)),
        out_shape=(jax.ShapeDtypeStruct((P, 1), f32),
                   jax.ShapeDtypeStruct((P, 1), f32)),
    )(sp16, sn16)


def kernel(input_feat, edge_index, edge_type, pos_edge_index, pos_edge_type,
           neg_dst, rel_emb, W_out, b_out, W_in, b_in, W1, b1, W2, b2):
    src_m = edge_index[0].reshape(NW, E // NW // K1, K1)
    dst_m = edge_index[1].reshape(NW, E // NW // K1, K1)
    et_m = edge_type.reshape(NW, E // NW // K1, K1)
    z2d = jnp.zeros((NP, D), jnp.float32)
    ones_tbl = jnp.ones((N, D), jnp.float32)
    rel_pad = jnp.concatenate(
        [rel_emb, jnp.zeros((N - NRELS, D), jnp.float32)], axis=0)

    cs = _gather_scatter(ones_tbl, src_m, src_m, z2d)
    cd = _gather_scatter(ones_tbl, dst_m, dst_m, z2d)
    ra = _gather_scatter(rel_pad, et_m, src_m, z2d)
    rb = _gather_scatter(rel_pad, et_m, dst_m, z2d)
    y1 = _fuse1(input_feat, ra[0], ra[1], rb[0], rb[1],
                cs[0], cs[1], cd[0], cd[1],
                W_out, b_out.reshape(1, D), W_in, b_in.reshape(1, D), W1)
    m = _gather_scatter(y1, src_m, dst_m, z2d)
    y2 = _fuse2(m[0], m[1], cs[0], cs[1], cd[0], cd[1], b1.reshape(1, D), W2)
    n = _gather_scatter(y2, src_m, dst_m, z2d)
    x2 = _fuse3(n[0], n[1], cd[0], cd[1], b2.reshape(1, D))

    ph3 = pos_edge_index[0].reshape(NW, P // NW // K3, K3)
    pt3 = pos_edge_index[1].reshape(NW, P // NW // K3, K3)
    pr3 = pos_edge_type.reshape(NW, P // NW // K3, K3)
    pn3 = neg_dst.reshape(NW, P // NW // K3, K3)
    sp16, sn16 = _score(x2, rel_emb, ph3, pt3, pr3, pn3)
    pos, neg = _sqrt2(sp16, sn16)
    return pos.reshape(P), neg.reshape(P)
